# baseline (speedup 1.0000x reference)
import functools

import jax
import jax.numpy as jnp
from jax import lax
from jax.experimental import pallas as pl
from jax.experimental.pallas import tpu as pltpu
from jax.experimental.pallas import tpu_sc as plsc

N = 10000
E = 320000
D = 128
DC = 8
NC = 2
NS = 16
NW = NC * NS
K = 96
NCHUNK = 105
EPAD = NW * NCHUNK * K
NPAD = N + 16
RPT = 624
ZTAIL = NPAD - NS * RPT
DTAIL = N - NS * RPT
BM = 2000


def _mm0_body(x_ref, wl_ref, wr_ref, table_ref, xr_ref):
    x = x_ref[...]
    table_ref[...] = jnp.dot(x, wl_ref[...], preferred_element_type=jnp.float32)
    xr_ref[...] = jnp.dot(x, wr_ref[...], preferred_element_type=jnp.float32)


def _mm1_body(s_ref, c_ref, xr_ref, wl_ref, wr_ref, table_ref, xr1_ref):
    ssum = s_ref[0] + s_ref[1]
    cnt = jnp.maximum(c_ref[0, :, :1] + c_ref[1, :, :1], 1.0)
    h = jnp.maximum(ssum / cnt + xr_ref[...], 0.0)
    table_ref[...] = jnp.dot(h, wl_ref[...], preferred_element_type=jnp.float32)
    xr1_ref[...] = jnp.dot(h, wr_ref[...], preferred_element_type=jnp.float32)


def _final_body(s_ref, c_ref, xr_ref, b_ref, out_ref):
    ssum = s_ref[0] + s_ref[1]
    cnt = jnp.maximum(c_ref[0, :, :1] + c_ref[1, :, :1], 1.0)
    out_ref[...] = ssum / cnt + xr_ref[...] + b_ref[...]


def _mm0(x, wl, wr):
    return pl.pallas_call(
        _mm0_body,
        grid=(N // BM,),
        in_specs=[
            pl.BlockSpec((BM, D), lambda i: (i, 0)),
            pl.BlockSpec((D, D), lambda i: (0, 0)),
            pl.BlockSpec((D, D), lambda i: (0, 0)),
        ],
        out_specs=[
            pl.BlockSpec((BM, D), lambda i: (i, 0)),
            pl.BlockSpec((BM, D), lambda i: (i, 0)),
        ],
        out_shape=[
            jax.ShapeDtypeStruct((N, D), jnp.float32),
            jax.ShapeDtypeStruct((N, D), jnp.float32),
        ],
    )(x, wl, wr)


def _mm1(s, c, xr, wl, wr):
    return pl.pallas_call(
        _mm1_body,
        grid=(N // BM,),
        in_specs=[
            pl.BlockSpec((NC, BM, D), lambda i: (0, i, 0)),
            pl.BlockSpec((NC, BM, DC), lambda i: (0, i, 0)),
            pl.BlockSpec((BM, D), lambda i: (i, 0)),
            pl.BlockSpec((D, D), lambda i: (0, 0)),
            pl.BlockSpec((D, D), lambda i: (0, 0)),
        ],
        out_specs=[
            pl.BlockSpec((BM, D), lambda i: (i, 0)),
            pl.BlockSpec((BM, D), lambda i: (i, 0)),
        ],
        out_shape=[
            jax.ShapeDtypeStruct((N, D), jnp.float32),
            jax.ShapeDtypeStruct((N, D), jnp.float32),
        ],
    )(s, c, xr, wl, wr)


def _final(s, c, xr, b):
    return pl.pallas_call(
        _final_body,
        grid=(N // BM,),
        in_specs=[
            pl.BlockSpec((NC, BM, D), lambda i: (0, i, 0)),
            pl.BlockSpec((NC, BM, DC), lambda i: (0, i, 0)),
            pl.BlockSpec((BM, D), lambda i: (i, 0)),
            pl.BlockSpec((1, D), lambda i: (0, 0)),
        ],
        out_specs=pl.BlockSpec((BM, D), lambda i: (i, 0)),
        out_shape=jax.ShapeDtypeStruct((N, D), jnp.float32),
    )(s, c, xr, b)


def _agg_refs(with_cnt):
    mesh = plsc.VectorSubcoreMesh(core_axis_name="c", subcore_axis_name="s")

    out_type = [jax.ShapeDtypeStruct((NC, N, D), jnp.float32)]
    scratch = [
        pltpu.VMEM((NCHUNK, K), jnp.int32),
        pltpu.VMEM((NCHUNK, K), jnp.int32),
        pltpu.VMEM((K, D), jnp.float32),
        pltpu.VMEM((K, D), jnp.float32),
        pltpu.VMEM_SHARED((NPAD, D), jnp.float32),
        pltpu.SemaphoreType.DMA,
        pltpu.SemaphoreType.DMA,
    ]
    if with_cnt:
        out_type.append(jax.ShapeDtypeStruct((NC, N, DC), jnp.float32))
        scratch.insert(4, pltpu.VMEM((K, DC), jnp.float32))
        scratch.insert(6, pltpu.VMEM_SHARED((NPAD, DC), jnp.float32))

    def body(*refs):
        if with_cnt:
            (table_hbm, src_hbm, dst_hbm, zeros_hbm, zeros8_hbm, ones8_hbm,
             out_hbm, outc_hbm,
             src_v, dst_v, rows0_v, rows1_v, ones_v, acc_sh, cnt_sh,
             sem0, sem1) = refs
        else:
            (table_hbm, src_hbm, dst_hbm, zeros_hbm,
             out_hbm,
             src_v, dst_v, rows0_v, rows1_v, acc_sh,
             sem0, sem1) = refs

        c = lax.axis_index("c")
        s = lax.axis_index("s")
        wid = c * NS + s

        pltpu.sync_copy(src_hbm.at[wid], src_v)
        pltpu.sync_copy(dst_hbm.at[wid], dst_v)
        if with_cnt:
            pltpu.sync_copy(ones8_hbm, ones_v)

        off = pl.multiple_of(s * RPT, 8)
        pltpu.sync_copy(zeros_hbm.at[pl.ds(off, RPT)],
                        acc_sh.at[pl.ds(off, RPT)])
        if with_cnt:
            pltpu.sync_copy(zeros8_hbm.at[pl.ds(off, RPT)],
                            cnt_sh.at[pl.ds(off, RPT)])

        @pl.when(s == NS - 1)
        def _zero_tail():
            pltpu.sync_copy(zeros_hbm.at[pl.ds(NS * RPT, ZTAIL)],
                            acc_sh.at[pl.ds(NS * RPT, ZTAIL)])
            if with_cnt:
                pltpu.sync_copy(zeros8_hbm.at[pl.ds(NS * RPT, ZTAIL)],
                                cnt_sh.at[pl.ds(NS * RPT, ZTAIL)])

        plsc.subcore_barrier()

        def gather(t, rows_v, sem):
            pltpu.async_copy(table_hbm.at[src_v.at[t]], rows_v, sem)

        def gwait(rows_v, sem):
            pltpu.make_async_copy(table_hbm.at[src_v.at[0]], rows_v, sem).wait()

        def scatter(t, rows_v):
            pltpu.sync_copy(rows_v, acc_sh.at[dst_v.at[t]], add=True)
            if with_cnt:
                pltpu.sync_copy(ones_v, cnt_sh.at[dst_v.at[t]], add=True)

        gather(0, rows0_v, sem0)

        def chunk2(i, carry):
            t = 2 * i
            gather(t + 1, rows1_v, sem1)
            gwait(rows0_v, sem0)
            scatter(t, rows0_v)

            @pl.when(t + 2 < NCHUNK)
            def _prefetch():
                gather(t + 2, rows0_v, sem0)

            gwait(rows1_v, sem1)
            scatter(t + 1, rows1_v)
            return carry

        lax.fori_loop(0, NCHUNK // 2, chunk2, 0)
        if NCHUNK % 2:
            gwait(rows0_v, sem0)
            scatter(NCHUNK - 1, rows0_v)
        plsc.subcore_barrier()

        off2 = pl.multiple_of(s * RPT, 8)
        pltpu.sync_copy(acc_sh.at[pl.ds(off2, RPT)],
                        out_hbm.at[c].at[pl.ds(off2, RPT)])
        if with_cnt:
            pltpu.sync_copy(cnt_sh.at[pl.ds(off2, RPT)],
                            outc_hbm.at[c].at[pl.ds(off2, RPT)])

        @pl.when(s == NS - 1)
        def _drain_tail():
            pltpu.sync_copy(acc_sh.at[pl.ds(NS * RPT, DTAIL)],
                            out_hbm.at[c].at[pl.ds(NS * RPT, DTAIL)])
            if with_cnt:
                pltpu.sync_copy(cnt_sh.at[pl.ds(NS * RPT, DTAIL)],
                                outc_hbm.at[c].at[pl.ds(NS * RPT, DTAIL)])

    kern = functools.partial(
        pl.kernel,
        out_type=out_type if with_cnt else out_type[0],
        mesh=mesh,
        scratch_types=scratch,
        compiler_params=pltpu.CompilerParams(use_tc_tiling_on_sc=False),
    )(body)
    return kern


_agg_cnt = _agg_refs(True)
_agg_plain = _agg_refs(False)


def kernel(x, edge_index, Wl0, Wr0, Wl1, Wr1, b1):
    src = edge_index[0].astype(jnp.int32)
    dst = edge_index[1].astype(jnp.int32)
    src = jnp.pad(src, (0, EPAD - E)).reshape(NW, NCHUNK, K)
    dst = jnp.pad(dst, (0, EPAD - E),
                  constant_values=N).reshape(NW, NCHUNK, K)
    zeros = jnp.zeros((NPAD, D), jnp.float32)
    zeros8 = jnp.zeros((NPAD, DC), jnp.float32)
    ones8 = jnp.ones((K, DC), jnp.float32)
    b = b1.reshape(1, D)

    table0, xr0 = _mm0(x, Wl0, Wr0)
    s0, c0 = _agg_cnt(table0, src, dst, zeros, zeros8, ones8)
    table1, xr1 = _mm1(s0, c0, xr0, Wl1, Wr1)
    s1 = _agg_plain(table1, src, dst, zeros)
    return _final(s1, c0, xr1, b)


# speedup vs baseline: 1.6609x; 1.6609x over previous
import functools

import jax
import jax.numpy as jnp
from jax import lax
from jax.experimental import pallas as pl
from jax.experimental.pallas import tpu as pltpu
from jax.experimental.pallas import tpu_sc as plsc

N = 10000
E = 320000
D = 128
DC = 8
NC = 2
NS = 16
NW = NC * NS
K = 80
NCHUNK = 125
NPAD = N + 16
RPT = 624
ZTAIL = NPAD - NS * RPT
DTAIL = N - NS * RPT
BM = 2000


def _mm0_body(x_ref, wl_ref, wr_ref, table_ref, xr_ref):
    x = x_ref[...]
    table_ref[...] = jnp.dot(x, wl_ref[...], preferred_element_type=jnp.float32)
    xr_ref[...] = jnp.dot(x, wr_ref[...], preferred_element_type=jnp.float32)


def _mm1_body(s_ref, c_ref, xr_ref, wl_ref, wr_ref, table_ref, xr1_ref):
    ssum = s_ref[0] + s_ref[1]
    cnt = jnp.maximum(c_ref[0, :, :1] + c_ref[1, :, :1], 1.0)
    h = jnp.maximum(ssum / cnt + xr_ref[...], 0.0)
    table_ref[...] = jnp.dot(h, wl_ref[...], preferred_element_type=jnp.float32)
    xr1_ref[...] = jnp.dot(h, wr_ref[...], preferred_element_type=jnp.float32)


def _final_body(s_ref, c_ref, xr_ref, b_ref, out_ref):
    ssum = s_ref[0] + s_ref[1]
    cnt = jnp.maximum(c_ref[0, :, :1] + c_ref[1, :, :1], 1.0)
    out_ref[...] = ssum / cnt + xr_ref[...] + b_ref[...]


def _mm0(x, wl, wr):
    return pl.pallas_call(
        _mm0_body,
        grid=(N // BM,),
        in_specs=[
            pl.BlockSpec((BM, D), lambda i: (i, 0)),
            pl.BlockSpec((D, D), lambda i: (0, 0)),
            pl.BlockSpec((D, D), lambda i: (0, 0)),
        ],
        out_specs=[
            pl.BlockSpec((BM, D), lambda i: (i, 0)),
            pl.BlockSpec((BM, D), lambda i: (i, 0)),
        ],
        out_shape=[
            jax.ShapeDtypeStruct((N, D), jnp.float32),
            jax.ShapeDtypeStruct((N, D), jnp.float32),
        ],
    )(x, wl, wr)


def _mm1(s, c, xr, wl, wr):
    return pl.pallas_call(
        _mm1_body,
        grid=(N // BM,),
        in_specs=[
            pl.BlockSpec((NC, BM, D), lambda i: (0, i, 0)),
            pl.BlockSpec((NC, BM, DC), lambda i: (0, i, 0)),
            pl.BlockSpec((BM, D), lambda i: (i, 0)),
            pl.BlockSpec((D, D), lambda i: (0, 0)),
            pl.BlockSpec((D, D), lambda i: (0, 0)),
        ],
        out_specs=[
            pl.BlockSpec((BM, D), lambda i: (i, 0)),
            pl.BlockSpec((BM, D), lambda i: (i, 0)),
        ],
        out_shape=[
            jax.ShapeDtypeStruct((N, D), jnp.float32),
            jax.ShapeDtypeStruct((N, D), jnp.float32),
        ],
    )(s, c, xr, wl, wr)


def _final(s, c, xr, b):
    return pl.pallas_call(
        _final_body,
        grid=(N // BM,),
        in_specs=[
            pl.BlockSpec((NC, BM, D), lambda i: (0, i, 0)),
            pl.BlockSpec((NC, BM, DC), lambda i: (0, i, 0)),
            pl.BlockSpec((BM, D), lambda i: (i, 0)),
            pl.BlockSpec((1, D), lambda i: (0, 0)),
        ],
        out_specs=pl.BlockSpec((BM, D), lambda i: (i, 0)),
        out_shape=jax.ShapeDtypeStruct((N, D), jnp.float32),
    )(s, c, xr, b)


def _agg_refs(with_cnt):
    mesh = plsc.VectorSubcoreMesh(core_axis_name="c", subcore_axis_name="s")

    out_type = [jax.ShapeDtypeStruct((NC, N, D), jnp.float32)]
    scratch = [
        pltpu.VMEM((NCHUNK, K), jnp.int32),
        pltpu.VMEM((NCHUNK, K), jnp.int32),
        pltpu.VMEM((K, D), jnp.float32),
        pltpu.VMEM((K, D), jnp.float32),
        pltpu.VMEM_SHARED((NPAD, D), jnp.float32),
        pltpu.SemaphoreType.DMA,
        pltpu.SemaphoreType.DMA,
    ]
    if with_cnt:
        out_type.append(jax.ShapeDtypeStruct((NC, N, DC), jnp.float32))
        scratch.insert(4, pltpu.VMEM((K, DC), jnp.float32))
        scratch.insert(6, pltpu.VMEM_SHARED((NPAD, DC), jnp.float32))

    def body(*refs):
        if with_cnt:
            (table_hbm, src_hbm, dst_hbm, zeros_hbm, zeros8_hbm, ones8_hbm,
             out_hbm, outc_hbm,
             src_v, dst_v, rows0_v, rows1_v, ones_v, acc_sh, cnt_sh,
             sem0, sem1) = refs
        else:
            (table_hbm, src_hbm, dst_hbm, zeros_hbm,
             out_hbm,
             src_v, dst_v, rows0_v, rows1_v, acc_sh,
             sem0, sem1) = refs

        c = lax.axis_index("c")
        s = lax.axis_index("s")
        wid = c * NS + s

        pltpu.sync_copy(src_hbm.at[wid], src_v)
        pltpu.sync_copy(dst_hbm.at[wid], dst_v)
        if with_cnt:
            pltpu.sync_copy(ones8_hbm, ones_v)

        off = pl.multiple_of(s * RPT, 8)
        pltpu.sync_copy(zeros_hbm.at[pl.ds(off, RPT)],
                        acc_sh.at[pl.ds(off, RPT)])
        if with_cnt:
            pltpu.sync_copy(zeros8_hbm.at[pl.ds(off, RPT)],
                            cnt_sh.at[pl.ds(off, RPT)])

        @pl.when(s == NS - 1)
        def _zero_tail():
            pltpu.sync_copy(zeros_hbm.at[pl.ds(NS * RPT, ZTAIL)],
                            acc_sh.at[pl.ds(NS * RPT, ZTAIL)])
            if with_cnt:
                pltpu.sync_copy(zeros8_hbm.at[pl.ds(NS * RPT, ZTAIL)],
                                cnt_sh.at[pl.ds(NS * RPT, ZTAIL)])

        plsc.subcore_barrier()

        def gather(t, rows_v, sem):
            pltpu.async_copy(table_hbm.at[src_v.at[t]], rows_v, sem)

        def gwait(rows_v, sem):
            pltpu.make_async_copy(table_hbm.at[src_v.at[0]], rows_v, sem).wait()

        def scatter(t, rows_v):
            pltpu.sync_copy(rows_v, acc_sh.at[dst_v.at[t]], add=True)
            if with_cnt:
                pltpu.sync_copy(ones_v, cnt_sh.at[dst_v.at[t]], add=True)

        gather(0, rows0_v, sem0)

        def chunk2(i, carry):
            t = 2 * i
            gather(t + 1, rows1_v, sem1)
            gwait(rows0_v, sem0)
            scatter(t, rows0_v)

            @pl.when(t + 2 < NCHUNK)
            def _prefetch():
                gather(t + 2, rows0_v, sem0)

            gwait(rows1_v, sem1)
            scatter(t + 1, rows1_v)
            return carry

        lax.fori_loop(0, NCHUNK // 2, chunk2, 0)
        if NCHUNK % 2:
            gwait(rows0_v, sem0)
            scatter(NCHUNK - 1, rows0_v)
        plsc.subcore_barrier()

        off2 = pl.multiple_of(s * RPT, 8)
        pltpu.sync_copy(acc_sh.at[pl.ds(off2, RPT)],
                        out_hbm.at[c].at[pl.ds(off2, RPT)])
        if with_cnt:
            pltpu.sync_copy(cnt_sh.at[pl.ds(off2, RPT)],
                            outc_hbm.at[c].at[pl.ds(off2, RPT)])

        @pl.when(s == NS - 1)
        def _drain_tail():
            pltpu.sync_copy(acc_sh.at[pl.ds(NS * RPT, DTAIL)],
                            out_hbm.at[c].at[pl.ds(NS * RPT, DTAIL)])
            if with_cnt:
                pltpu.sync_copy(cnt_sh.at[pl.ds(NS * RPT, DTAIL)],
                                outc_hbm.at[c].at[pl.ds(NS * RPT, DTAIL)])

    kern = functools.partial(
        pl.kernel,
        out_type=out_type if with_cnt else out_type[0],
        mesh=mesh,
        scratch_types=scratch,
        compiler_params=pltpu.CompilerParams(use_tc_tiling_on_sc=False),
    )(body)
    return kern


_agg_cnt = _agg_refs(True)
_agg_plain = _agg_refs(False)


def kernel(x, edge_index, Wl0, Wr0, Wl1, Wr1, b1):
    src = edge_index[0].astype(jnp.int32)
    dst = edge_index[1].astype(jnp.int32)
    src = src.reshape(NW, NCHUNK, K)
    dst = dst.reshape(NW, NCHUNK, K)
    zeros = jnp.zeros((NPAD, D), jnp.float32)
    zeros8 = jnp.zeros((NPAD, DC), jnp.float32)
    ones8 = jnp.ones((K, DC), jnp.float32)
    b = b1.reshape(1, D)

    table0, xr0 = _mm0(x, Wl0, Wr0)
    s0, c0 = _agg_cnt(table0, src, dst, zeros, zeros8, ones8)
    table1, xr1 = _mm1(s0, c0, xr0, Wl1, Wr1)
    s1 = _agg_plain(table1, src, dst, zeros)
    return _final(s1, c0, xr1, b)


# speedup vs baseline: 1.7043x; 1.0262x over previous
import functools

import jax
import jax.numpy as jnp
from jax import lax
from jax.experimental import pallas as pl
from jax.experimental.pallas import tpu as pltpu
from jax.experimental.pallas import tpu_sc as plsc

N = 10000
E = 320000
D = 128
DC = 8
NC = 2
NS = 16
NW = NC * NS
K = 80
NCHUNK = 125
NPAD = N + 16
RPT = 624
ZTAIL = NPAD - NS * RPT
DTAIL = N - NS * RPT
BM = 2000


def _mm0_body(x_ref, wl_ref, wr_ref, table_ref, xr_ref):
    x = x_ref[...]
    yl = jnp.dot(x, wl_ref[...], preferred_element_type=jnp.float32)
    table_ref[...] = yl.astype(jnp.bfloat16)
    xr_ref[...] = jnp.dot(x, wr_ref[...], preferred_element_type=jnp.float32)


def _mm1_body(s_ref, c_ref, xr_ref, wl_ref, wr_ref, table_ref, xr1_ref):
    ssum = s_ref[0].astype(jnp.float32) + s_ref[1].astype(jnp.float32)
    cnt = jnp.maximum(c_ref[0, :, :1] + c_ref[1, :, :1], 1.0)
    h = jnp.maximum(ssum / cnt + xr_ref[...], 0.0)
    yl = jnp.dot(h, wl_ref[...], preferred_element_type=jnp.float32)
    table_ref[...] = yl.astype(jnp.bfloat16)
    xr1_ref[...] = jnp.dot(h, wr_ref[...], preferred_element_type=jnp.float32)


def _final_body(s_ref, c_ref, xr_ref, b_ref, out_ref):
    ssum = s_ref[0].astype(jnp.float32) + s_ref[1].astype(jnp.float32)
    cnt = jnp.maximum(c_ref[0, :, :1] + c_ref[1, :, :1], 1.0)
    out_ref[...] = ssum / cnt + xr_ref[...] + b_ref[...]


def _mm0(x, wl, wr):
    return pl.pallas_call(
        _mm0_body,
        grid=(N // BM,),
        in_specs=[
            pl.BlockSpec((BM, D), lambda i: (i, 0)),
            pl.BlockSpec((D, D), lambda i: (0, 0)),
            pl.BlockSpec((D, D), lambda i: (0, 0)),
        ],
        out_specs=[
            pl.BlockSpec((BM, D), lambda i: (i, 0)),
            pl.BlockSpec((BM, D), lambda i: (i, 0)),
        ],
        out_shape=[
            jax.ShapeDtypeStruct((N, D), jnp.bfloat16),
            jax.ShapeDtypeStruct((N, D), jnp.float32),
        ],
    )(x, wl, wr)


def _mm1(s, c, xr, wl, wr):
    return pl.pallas_call(
        _mm1_body,
        grid=(N // BM,),
        in_specs=[
            pl.BlockSpec((NC, BM, D), lambda i: (0, i, 0)),
            pl.BlockSpec((NC, BM, DC), lambda i: (0, i, 0)),
            pl.BlockSpec((BM, D), lambda i: (i, 0)),
            pl.BlockSpec((D, D), lambda i: (0, 0)),
            pl.BlockSpec((D, D), lambda i: (0, 0)),
        ],
        out_specs=[
            pl.BlockSpec((BM, D), lambda i: (i, 0)),
            pl.BlockSpec((BM, D), lambda i: (i, 0)),
        ],
        out_shape=[
            jax.ShapeDtypeStruct((N, D), jnp.bfloat16),
            jax.ShapeDtypeStruct((N, D), jnp.float32),
        ],
    )(s, c, xr, wl, wr)


def _final(s, c, xr, b):
    return pl.pallas_call(
        _final_body,
        grid=(N // BM,),
        in_specs=[
            pl.BlockSpec((NC, BM, D), lambda i: (0, i, 0)),
            pl.BlockSpec((NC, BM, DC), lambda i: (0, i, 0)),
            pl.BlockSpec((BM, D), lambda i: (i, 0)),
            pl.BlockSpec((1, D), lambda i: (0, 0)),
        ],
        out_specs=pl.BlockSpec((BM, D), lambda i: (i, 0)),
        out_shape=jax.ShapeDtypeStruct((N, D), jnp.float32),
    )(s, c, xr, b)


def _agg_refs(with_cnt):
    mesh = plsc.VectorSubcoreMesh(core_axis_name="c", subcore_axis_name="s")

    out_type = [jax.ShapeDtypeStruct((NC, N, D), jnp.bfloat16)]
    scratch = [
        pltpu.VMEM((NCHUNK, K), jnp.int32),
        pltpu.VMEM((NCHUNK, K), jnp.int32),
        pltpu.VMEM((K, D), jnp.bfloat16),
        pltpu.VMEM((K, D), jnp.bfloat16),
        pltpu.VMEM_SHARED((NPAD, D), jnp.bfloat16),
        pltpu.SemaphoreType.DMA,
        pltpu.SemaphoreType.DMA,
    ]
    if with_cnt:
        out_type.append(jax.ShapeDtypeStruct((NC, N, DC), jnp.float32))
        scratch.insert(4, pltpu.VMEM((K, DC), jnp.float32))
        scratch.insert(6, pltpu.VMEM_SHARED((NPAD, DC), jnp.float32))

    def body(*refs):
        if with_cnt:
            (table_hbm, src_hbm, dst_hbm, zeros_hbm, zeros8_hbm, ones8_hbm,
             out_hbm, outc_hbm,
             src_v, dst_v, rows0_v, rows1_v, ones_v, acc_sh, cnt_sh,
             sem0, sem1) = refs
        else:
            (table_hbm, src_hbm, dst_hbm, zeros_hbm,
             out_hbm,
             src_v, dst_v, rows0_v, rows1_v, acc_sh,
             sem0, sem1) = refs

        c = lax.axis_index("c")
        s = lax.axis_index("s")
        wid = c * NS + s

        pltpu.sync_copy(src_hbm.at[wid], src_v)
        pltpu.sync_copy(dst_hbm.at[wid], dst_v)
        if with_cnt:
            pltpu.sync_copy(ones8_hbm, ones_v)

        off = pl.multiple_of(s * RPT, 8)
        pltpu.sync_copy(zeros_hbm.at[pl.ds(off, RPT)],
                        acc_sh.at[pl.ds(off, RPT)])
        if with_cnt:
            pltpu.sync_copy(zeros8_hbm.at[pl.ds(off, RPT)],
                            cnt_sh.at[pl.ds(off, RPT)])

        @pl.when(s == NS - 1)
        def _zero_tail():
            pltpu.sync_copy(zeros_hbm.at[pl.ds(NS * RPT, ZTAIL)],
                            acc_sh.at[pl.ds(NS * RPT, ZTAIL)])
            if with_cnt:
                pltpu.sync_copy(zeros8_hbm.at[pl.ds(NS * RPT, ZTAIL)],
                                cnt_sh.at[pl.ds(NS * RPT, ZTAIL)])

        plsc.subcore_barrier()

        def gather(t, rows_v, sem):
            pltpu.async_copy(table_hbm.at[src_v.at[t]], rows_v, sem)

        def gwait(rows_v, sem):
            pltpu.make_async_copy(table_hbm.at[src_v.at[0]], rows_v, sem).wait()

        def scatter(t, rows_v):
            pltpu.sync_copy(rows_v, acc_sh.at[dst_v.at[t]], add=True)
            if with_cnt:
                pltpu.sync_copy(ones_v, cnt_sh.at[dst_v.at[t]], add=True)

        gather(0, rows0_v, sem0)

        def chunk2(i, carry):
            t = 2 * i
            gather(t + 1, rows1_v, sem1)
            gwait(rows0_v, sem0)
            scatter(t, rows0_v)

            @pl.when(t + 2 < NCHUNK)
            def _prefetch():
                gather(t + 2, rows0_v, sem0)

            gwait(rows1_v, sem1)
            scatter(t + 1, rows1_v)
            return carry

        lax.fori_loop(0, NCHUNK // 2, chunk2, 0)
        if NCHUNK % 2:
            gwait(rows0_v, sem0)
            scatter(NCHUNK - 1, rows0_v)
        plsc.subcore_barrier()

        off2 = pl.multiple_of(s * RPT, 8)
        pltpu.sync_copy(acc_sh.at[pl.ds(off2, RPT)],
                        out_hbm.at[c].at[pl.ds(off2, RPT)])
        if with_cnt:
            pltpu.sync_copy(cnt_sh.at[pl.ds(off2, RPT)],
                            outc_hbm.at[c].at[pl.ds(off2, RPT)])

        @pl.when(s == NS - 1)
        def _drain_tail():
            pltpu.sync_copy(acc_sh.at[pl.ds(NS * RPT, DTAIL)],
                            out_hbm.at[c].at[pl.ds(NS * RPT, DTAIL)])
            if with_cnt:
                pltpu.sync_copy(cnt_sh.at[pl.ds(NS * RPT, DTAIL)],
                                outc_hbm.at[c].at[pl.ds(NS * RPT, DTAIL)])

    kern = functools.partial(
        pl.kernel,
        out_type=out_type if with_cnt else out_type[0],
        mesh=mesh,
        scratch_types=scratch,
        compiler_params=pltpu.CompilerParams(use_tc_tiling_on_sc=False),
    )(body)
    return kern


_agg_cnt = _agg_refs(True)
_agg_plain = _agg_refs(False)


def kernel(x, edge_index, Wl0, Wr0, Wl1, Wr1, b1):
    src = edge_index[0].astype(jnp.int32)
    dst = edge_index[1].astype(jnp.int32)
    src = src.reshape(NW, NCHUNK, K)
    dst = dst.reshape(NW, NCHUNK, K)
    zeros = jnp.zeros((NPAD, D), jnp.bfloat16)
    zeros8 = jnp.zeros((NPAD, DC), jnp.float32)
    ones8 = jnp.ones((K, DC), jnp.float32)
    b = b1.reshape(1, D)

    table0, xr0 = _mm0(x, Wl0, Wr0)
    s0, c0 = _agg_cnt(table0, src, dst, zeros, zeros8, ones8)
    table1, xr1 = _mm1(s0, c0, xr0, Wl1, Wr1)
    s1 = _agg_plain(table1, src, dst, zeros)
    return _final(s1, c0, xr1, b)
